# baseline (speedup 1.0000x reference)
import functools

import jax
import jax.numpy as jnp
from jax import lax
from jax.experimental import pallas as pl
from jax.experimental.pallas import tpu as pltpu

_R = 16


@functools.lru_cache
def _build(B, V, D):
    G = B // _R

    def tspec(j):
        return pl.BlockSpec((8, D), lambda i, idx: (idx[i * _R + j] // 8, 0))

    grid_spec = pltpu.PrefetchScalarGridSpec(
        num_scalar_prefetch=1,
        grid=(G,),
        in_specs=[tspec(j) for j in range(_R)],
        out_specs=pl.BlockSpec((_R, D), lambda i, idx: (i, 0)),
    )

    def body(idx_ref, *refs):
        out_ref = refs[_R]
        i = pl.program_id(0)
        rows = []
        for j in range(_R):
            lo = idx_ref[i * _R + j] % 8
            bj = refs[j][...]
            mask = lax.broadcasted_iota(jnp.int32, (8, D), 0) == lo
            rows.append(jnp.sum(jnp.where(mask, bj, 0.0), axis=0, keepdims=True))
        out_ref[...] = jnp.concatenate(rows, axis=0)

    return pl.pallas_call(
        body,
        grid_spec=grid_spec,
        out_shape=jax.ShapeDtypeStruct((B, D), jnp.float32),
    )


def kernel(cells, w_cell_emb):
    B, = cells.shape
    V, D = w_cell_emb.shape
    return _build(B, V, D)(cells.astype(jnp.int32), *([w_cell_emb] * _R))


# speedup vs baseline: 2.3427x; 2.3427x over previous
import functools

import jax
import jax.numpy as jnp
from jax import lax
from jax.experimental import pallas as pl
from jax.experimental.pallas import tpu as pltpu

_CH = 512
_UNROLL = 8


@functools.lru_cache
def _build(B, V, D):
    G = B // _CH

    grid_spec = pltpu.PrefetchScalarGridSpec(
        num_scalar_prefetch=1,
        grid=(G,),
        in_specs=[pl.BlockSpec(memory_space=pl.ANY)],
        out_specs=pl.BlockSpec((_CH, D), lambda i, idx: (i, 0)),
        scratch_shapes=[pltpu.SemaphoreType.DMA],
    )

    def body(idx_ref, table_ref, out_ref, sem):
        i = pl.program_id(0)
        base = i * _CH

        def fire(g, carry):
            for jj in range(_UNROLL):
                j = g * _UNROLL + jj
                row = idx_ref[base + j]
                pltpu.make_async_copy(
                    table_ref.at[pl.ds(row, 1)],
                    out_ref.at[pl.ds(j, 1)],
                    sem,
                ).start()
            return carry

        lax.fori_loop(0, _CH // _UNROLL, fire, 0, unroll=False)

        def drain(g, carry):
            for jj in range(_UNROLL):
                j = g * _UNROLL + jj
                pltpu.make_async_copy(
                    table_ref.at[pl.ds(0, 1)],
                    out_ref.at[pl.ds(j, 1)],
                    sem,
                ).wait()
            return carry

        lax.fori_loop(0, _CH // _UNROLL, drain, 0, unroll=False)

    return pl.pallas_call(
        body,
        grid_spec=grid_spec,
        out_shape=jax.ShapeDtypeStruct((B, D), jnp.float32),
    )


def kernel(cells, w_cell_emb):
    B, = cells.shape
    V, D = w_cell_emb.shape
    return _build(B, V, D)(cells.astype(jnp.int32), w_cell_emb)
